# baseline (speedup 1.0000x reference)
import functools

import jax
import jax.numpy as jnp
from jax import lax
from jax.experimental import pallas as pl
from jax.experimental.pallas import tpu as pltpu
from jax.experimental.pallas import tpu_sc as plsc

N_BINS = 15
N = 1048576
L = 16
NC, NS = 2, 16
NW = NC * NS
CHUNK = N // NW
BLK = 16384
NB = CHUNK // BLK
VECS = BLK // L
PROW = 48

_mesh = plsc.VectorSubcoreMesh(
    core_axis_name="c", subcore_axis_name="s", num_cores=NC, num_subcores=NS
)
_params = pltpu.CompilerParams(needs_layout_passes=False)


@functools.partial(
    pl.kernel,
    out_type=(
        jax.ShapeDtypeStruct((NC * PROW,), jnp.float32),
        jax.ShapeDtypeStruct((L,), jnp.float32),
    ),
    mesh=_mesh,
    compiler_params=_params,
    scratch_types=[
        pltpu.VMEM((2, BLK), jnp.int32),
        pltpu.VMEM((2, BLK), jnp.float32),
        pltpu.VMEM((2, BLK), jnp.int32),
        pltpu.VMEM((L,), jnp.float32),
        pltpu.VMEM((L * L,), jnp.float32),
        pltpu.VMEM((L * L,), jnp.float32),
        pltpu.VMEM((PROW,), jnp.float32),
        pltpu.VMEM_SHARED((NS * PROW,), jnp.float32),
        pltpu.VMEM((NS * PROW,), jnp.float32),
        pltpu.VMEM((PROW,), jnp.float32),
        pltpu.VMEM((L,), jnp.float32),
        pltpu.SemaphoreType.DMA,
        pltpu.SemaphoreType.DMA,
        pltpu.SemaphoreType.REGULAR,
    ],
)
def _ece_hist_kernel(yhs_hbm, phs_hbm, ys_hbm, bounds_hbm,
                     rows_hbm, ece_hbm,
                     yv, pv, vv, btab, cc, cnf, row,
                     board, boardv, other, outv, sem0, sem1, xsem):
    cidx = lax.axis_index("c")
    sidx = lax.axis_index("s")
    wid = cidx * NS + sidx
    sems = [sem0, sem1]

    pending = []
    for b in range(NB):
        base = wid * CHUNK + b * BLK
        sem = sems[b]
        pending.append([
            pltpu.async_copy(yhs_hbm.at[pl.ds(base, BLK)], yv.at[b], sem),
            pltpu.async_copy(phs_hbm.at[pl.ds(base, BLK)], pv.at[b], sem),
            pltpu.async_copy(ys_hbm.at[pl.ds(base, BLK)], vv.at[b], sem),
        ])

    pltpu.sync_copy(bounds_hbm, btab)
    bvec = btab[...]

    zeros = jnp.zeros((L,), jnp.float32)

    def zero_bin(v, carry):
        cc[pl.ds(v * L, L)] = zeros
        cnf[pl.ds(v * L, L)] = zeros
        return carry

    lax.fori_loop(0, N_BINS, zero_bin, 0)

    lane = lax.iota(jnp.int32, L)

    for b in range(NB):
        for h in pending[b]:
            h.wait()
        s = b

        @plsc.parallel_loop(0, VECS, unroll=4)
        def _(i):
            off = i * L
            p = pv[s, pl.ds(off, L)]
            yh = yv[s, pl.ds(off, L)]
            yy = vv[s, pl.ds(off, L)]
            t2 = p * jnp.float32(N_BINS) + jnp.float32(0.5)
            j1 = t2.astype(jnp.int32)
            b1 = jnp.take_along_axis(bvec, j1, axis=0)
            j = j1 - (p <= b1).astype(jnp.int32)
            valid = p > jnp.float32(0.0)
            idx = j * L + lane
            packed = jnp.where(yh == yy, jnp.float32(4097.0), jnp.float32(4096.0))
            plsc.addupdate_scatter(cc, [idx], packed, mask=valid)
            plsc.addupdate_scatter(cnf, [idx], p, mask=valid)

    inv_k = jnp.float32(1.0 / 4096.0)
    kf = jnp.float32(4096.0)

    def fold_bin(j, carry):
        cnt_vec, cor_vec, cnf_vec = carry
        sel = lane == j
        s_cc = cc[pl.ds(j * L, L)]
        c_slot = (s_cc * inv_k).astype(jnp.int32).astype(jnp.float32)
        r_slot = s_cc - kf * c_slot
        cj = jnp.sum(c_slot)
        aj = jnp.sum(r_slot)
        fj = jnp.sum(cnf[pl.ds(j * L, L)])
        cnt_vec = jnp.where(sel, jnp.broadcast_to(cj, (L,)), cnt_vec)
        cor_vec = jnp.where(sel, jnp.broadcast_to(aj, (L,)), cor_vec)
        cnf_vec = jnp.where(sel, jnp.broadcast_to(fj, (L,)), cnf_vec)
        return (cnt_vec, cor_vec, cnf_vec)

    cnt_vec, cor_vec, cnf_vec = lax.fori_loop(
        0, N_BINS, fold_bin, (zeros, zeros, zeros)
    )
    row[pl.ds(0, L)] = cnt_vec
    row[pl.ds(L, L)] = cor_vec
    row[pl.ds(2 * L, L)] = cnf_vec

    pltpu.sync_copy(row, board.at[pl.ds(sidx * PROW, PROW)])
    plsc.subcore_barrier()

    @pl.when(sidx == 0)
    def _():
        pltpu.sync_copy(board, boardv)

        def add_tile(t, carry):
            c, a, f = carry
            o = t * PROW
            return (c + boardv[pl.ds(o, L)],
                    a + boardv[pl.ds(o + L, L)],
                    f + boardv[pl.ds(o + 2 * L, L)])

        c_sum, a_sum, f_sum = lax.fori_loop(
            0, NS, add_tile, (zeros, zeros, zeros)
        )
        row[pl.ds(0, L)] = c_sum
        row[pl.ds(L, L)] = a_sum
        row[pl.ds(2 * L, L)] = f_sum
        pltpu.sync_copy(row, rows_hbm.at[pl.ds(cidx * PROW, PROW)])
        pl.semaphore_signal(xsem, 1, device_id={"c": 1 - cidx, "s": 0})
        pl.semaphore_wait(xsem, 1)
        pltpu.sync_copy(rows_hbm.at[pl.ds((1 - cidx) * PROW, PROW)], other)

        counts_v = c_sum + other[pl.ds(0, L)]
        acc_v = a_sum + other[pl.ds(L, L)]
        conf_v = f_sum + other[pl.ds(2 * L, L)]
        ones = jnp.ones((L,), jnp.float32)
        ind = counts_v > jnp.float32(0.0)
        safe = jnp.where(ind, counts_v, ones)
        mean_acc = jnp.where(ind, acc_v / safe, acc_v)
        mean_conf = jnp.where(ind, conf_v / safe, conf_v)
        num = jnp.sum(counts_v * jnp.abs(mean_acc - mean_conf))
        tot = jnp.sum(counts_v)
        outv[...] = jnp.broadcast_to(num, (L,)) / jnp.broadcast_to(tot, (L,))

        @pl.when(cidx == 0)
        def _():
            pltpu.sync_copy(outv, ece_hbm)


@jax.jit
def kernel(yhs, phs, ys):
    bounds = jnp.linspace(0.0, 1.0, N_BINS + 1).astype(jnp.float32)
    _, ece_vec = _ece_hist_kernel(yhs, phs, ys, bounds)
    return ece_vec[0]


# speedup vs baseline: 1.0058x; 1.0058x over previous
import functools

import jax
import jax.numpy as jnp
from jax import lax
from jax.experimental import pallas as pl
from jax.experimental.pallas import tpu as pltpu
from jax.experimental.pallas import tpu_sc as plsc

N_BINS = 15
N = 1048576
L = 16
NC, NS = 2, 16
NW = NC * NS
CHUNK = N // NW
BLK = 16384
NB = CHUNK // BLK
VECS = BLK // L
PROW = 48

_mesh = plsc.VectorSubcoreMesh(
    core_axis_name="c", subcore_axis_name="s", num_cores=NC, num_subcores=NS
)
_params = pltpu.CompilerParams(needs_layout_passes=False)


@functools.partial(
    pl.kernel,
    out_type=(
        jax.ShapeDtypeStruct((NC * PROW,), jnp.float32),
        jax.ShapeDtypeStruct((L,), jnp.float32),
    ),
    mesh=_mesh,
    compiler_params=_params,
    scratch_types=[
        pltpu.VMEM((2, BLK), jnp.int32),
        pltpu.VMEM((2, BLK), jnp.float32),
        pltpu.VMEM((2, BLK), jnp.int32),
        pltpu.VMEM((L,), jnp.float32),
        pltpu.VMEM((L * L,), jnp.float32),
        pltpu.VMEM((L * L,), jnp.float32),
        pltpu.VMEM((PROW,), jnp.float32),
        pltpu.VMEM_SHARED((NS * PROW,), jnp.float32),
        pltpu.VMEM((NS * PROW,), jnp.float32),
        pltpu.VMEM((PROW,), jnp.float32),
        pltpu.VMEM((L,), jnp.float32),
        pltpu.SemaphoreType.DMA,
        pltpu.SemaphoreType.DMA,
        pltpu.SemaphoreType.REGULAR,
    ],
)
def _ece_hist_kernel(yhs_hbm, phs_hbm, ys_hbm, bounds_hbm,
                     rows_hbm, ece_hbm,
                     yv, pv, vv, btab, cc, cnf, row,
                     board, boardv, other, outv, sem0, sem1, xsem):
    cidx = lax.axis_index("c")
    sidx = lax.axis_index("s")
    wid = cidx * NS + sidx
    sems = [sem0, sem1]

    pltpu.sync_copy(bounds_hbm, btab)
    bvec = btab[...]

    pending = []
    for b in range(NB):
        base = wid * CHUNK + b * BLK
        sem = sems[b]
        pending.append([
            pltpu.async_copy(yhs_hbm.at[pl.ds(base, BLK)], yv.at[b], sem),
            pltpu.async_copy(phs_hbm.at[pl.ds(base, BLK)], pv.at[b], sem),
            pltpu.async_copy(ys_hbm.at[pl.ds(base, BLK)], vv.at[b], sem),
        ])

    zeros = jnp.zeros((L,), jnp.float32)

    def zero_bin(v, carry):
        cc[pl.ds(v * L, L)] = zeros
        cnf[pl.ds(v * L, L)] = zeros
        return carry

    lax.fori_loop(0, N_BINS, zero_bin, 0)

    lane = lax.iota(jnp.int32, L)

    for b in range(NB):
        for h in pending[b]:
            h.wait()
        s = b

        @plsc.parallel_loop(0, VECS, unroll=4)
        def _(i):
            off = i * L
            p = pv[s, pl.ds(off, L)]
            yh = yv[s, pl.ds(off, L)]
            yy = vv[s, pl.ds(off, L)]
            t2 = p * jnp.float32(N_BINS) + jnp.float32(0.5)
            j1 = t2.astype(jnp.int32)
            b1 = jnp.take_along_axis(bvec, j1, axis=0)
            j = j1 - (p <= b1).astype(jnp.int32)
            valid = p > jnp.float32(0.0)
            idx = j * L + lane
            packed = jnp.where(yh == yy, jnp.float32(4097.0), jnp.float32(4096.0))
            plsc.addupdate_scatter(cc, [idx], packed, mask=valid)
            plsc.addupdate_scatter(cnf, [idx], p, mask=valid)

    inv_k = jnp.float32(1.0 / 4096.0)
    kf = jnp.float32(4096.0)

    def fold_bin(j, carry):
        cnt_vec, cor_vec, cnf_vec = carry
        sel = lane == j
        s_cc = cc[pl.ds(j * L, L)]
        c_slot = (s_cc * inv_k).astype(jnp.int32).astype(jnp.float32)
        r_slot = s_cc - kf * c_slot
        cj = jnp.sum(c_slot)
        aj = jnp.sum(r_slot)
        fj = jnp.sum(cnf[pl.ds(j * L, L)])
        cnt_vec = jnp.where(sel, jnp.broadcast_to(cj, (L,)), cnt_vec)
        cor_vec = jnp.where(sel, jnp.broadcast_to(aj, (L,)), cor_vec)
        cnf_vec = jnp.where(sel, jnp.broadcast_to(fj, (L,)), cnf_vec)
        return (cnt_vec, cor_vec, cnf_vec)

    cnt_vec, cor_vec, cnf_vec = lax.fori_loop(
        0, N_BINS, fold_bin, (zeros, zeros, zeros)
    )
    row[pl.ds(0, L)] = cnt_vec
    row[pl.ds(L, L)] = cor_vec
    row[pl.ds(2 * L, L)] = cnf_vec

    pltpu.sync_copy(row, board.at[pl.ds(sidx * PROW, PROW)])
    plsc.subcore_barrier()

    @pl.when(sidx == 0)
    def _():
        pltpu.sync_copy(board, boardv)

        def add_tile(t, carry):
            c, a, f = carry
            o = t * PROW
            return (c + boardv[pl.ds(o, L)],
                    a + boardv[pl.ds(o + L, L)],
                    f + boardv[pl.ds(o + 2 * L, L)])

        c_sum, a_sum, f_sum = lax.fori_loop(
            0, NS, add_tile, (zeros, zeros, zeros)
        )
        row[pl.ds(0, L)] = c_sum
        row[pl.ds(L, L)] = a_sum
        row[pl.ds(2 * L, L)] = f_sum
        pltpu.sync_copy(row, rows_hbm.at[pl.ds(cidx * PROW, PROW)])
        pl.semaphore_signal(xsem, 1, device_id={"c": 1 - cidx, "s": 0})
        pl.semaphore_wait(xsem, 1)
        pltpu.sync_copy(rows_hbm.at[pl.ds((1 - cidx) * PROW, PROW)], other)

        counts_v = c_sum + other[pl.ds(0, L)]
        acc_v = a_sum + other[pl.ds(L, L)]
        conf_v = f_sum + other[pl.ds(2 * L, L)]
        ones = jnp.ones((L,), jnp.float32)
        ind = counts_v > jnp.float32(0.0)
        safe = jnp.where(ind, counts_v, ones)
        mean_acc = jnp.where(ind, acc_v / safe, acc_v)
        mean_conf = jnp.where(ind, conf_v / safe, conf_v)
        num = jnp.sum(counts_v * jnp.abs(mean_acc - mean_conf))
        tot = jnp.sum(counts_v)
        outv[...] = jnp.broadcast_to(num, (L,)) / jnp.broadcast_to(tot, (L,))

        @pl.when(cidx == 0)
        def _():
            pltpu.sync_copy(outv, ece_hbm)


@jax.jit
def kernel(yhs, phs, ys):
    bounds = jnp.linspace(0.0, 1.0, N_BINS + 1).astype(jnp.float32)
    _, ece_vec = _ece_hist_kernel(yhs, phs, ys, bounds)
    return ece_vec[0]


# speedup vs baseline: 1.0134x; 1.0075x over previous
import functools

import jax
import jax.numpy as jnp
from jax import lax
from jax.experimental import pallas as pl
from jax.experimental.pallas import tpu as pltpu
from jax.experimental.pallas import tpu_sc as plsc

N_BINS = 15
N = 1048576
L = 16
NC, NS = 2, 16
NW = NC * NS
CHUNK = N // NW
BLK = 16384
NB = CHUNK // BLK
VECS = BLK // L
PROW = 48

_mesh = plsc.VectorSubcoreMesh(
    core_axis_name="c", subcore_axis_name="s", num_cores=NC, num_subcores=NS
)
_params = pltpu.CompilerParams(needs_layout_passes=False)


@functools.partial(
    pl.kernel,
    out_type=(
        jax.ShapeDtypeStruct((NC * PROW,), jnp.float32),
        jax.ShapeDtypeStruct((L,), jnp.float32),
    ),
    mesh=_mesh,
    compiler_params=_params,
    scratch_types=[
        pltpu.VMEM((2, BLK), jnp.int32),
        pltpu.VMEM((2, BLK), jnp.float32),
        pltpu.VMEM((2, BLK), jnp.int32),
        pltpu.VMEM((L,), jnp.float32),
        pltpu.VMEM((L * L,), jnp.float32),
        pltpu.VMEM((L * L,), jnp.float32),
        pltpu.VMEM((PROW,), jnp.float32),
        pltpu.VMEM_SHARED((NS * PROW,), jnp.float32),
        pltpu.VMEM((NS * PROW,), jnp.float32),
        pltpu.VMEM((PROW,), jnp.float32),
        pltpu.VMEM((L,), jnp.float32),
        pltpu.SemaphoreType.DMA,
        pltpu.SemaphoreType.DMA,
        pltpu.SemaphoreType.REGULAR,
    ],
)
def _ece_hist_kernel(yhs_hbm, phs_hbm, ys_hbm, bounds_hbm,
                     rows_hbm, ece_hbm,
                     yv, pv, vv, btab, cc, cnf, row,
                     board, boardv, other, outv, sem0, sem1, xsem):
    cidx = lax.axis_index("c")
    sidx = lax.axis_index("s")
    wid = cidx * NS + sidx
    sems = [sem0, sem1]

    pltpu.sync_copy(bounds_hbm, btab)
    bvec = btab[...]

    pending = []
    for b in range(NB):
        base = wid * CHUNK + b * BLK
        sem = sems[b]
        pending.append([
            pltpu.async_copy(yhs_hbm.at[pl.ds(base, BLK)], yv.at[b], sem),
            pltpu.async_copy(phs_hbm.at[pl.ds(base, BLK)], pv.at[b], sem),
            pltpu.async_copy(ys_hbm.at[pl.ds(base, BLK)], vv.at[b], sem),
        ])

    zeros = jnp.zeros((L,), jnp.float32)

    def zero_bin(v, carry):
        cc[pl.ds(v * L, L)] = zeros
        cnf[pl.ds(v * L, L)] = zeros
        return carry

    lax.fori_loop(0, N_BINS, zero_bin, 0)

    lane = lax.iota(jnp.int32, L)

    for b in range(NB):
        for h in pending[b]:
            h.wait()
        s = b

        @plsc.parallel_loop(0, VECS, unroll=6)
        def _(i):
            off = i * L
            p = pv[s, pl.ds(off, L)]
            yh = yv[s, pl.ds(off, L)]
            yy = vv[s, pl.ds(off, L)]
            t2 = p * jnp.float32(N_BINS) + jnp.float32(0.5)
            j1 = t2.astype(jnp.int32)
            b1 = jnp.take_along_axis(bvec, j1, axis=0)
            j = j1 - (p <= b1).astype(jnp.int32)
            valid = p > jnp.float32(0.0)
            idx = j * L + lane
            packed = jnp.where(yh == yy, jnp.float32(4097.0), jnp.float32(4096.0))
            plsc.addupdate_scatter(cc, [idx], packed, mask=valid)
            plsc.addupdate_scatter(cnf, [idx], p, mask=valid)

    inv_k = jnp.float32(1.0 / 4096.0)
    kf = jnp.float32(4096.0)

    def fold_bin(j, carry):
        cnt_vec, cor_vec, cnf_vec = carry
        sel = lane == j
        s_cc = cc[pl.ds(j * L, L)]
        c_slot = (s_cc * inv_k).astype(jnp.int32).astype(jnp.float32)
        r_slot = s_cc - kf * c_slot
        cj = jnp.sum(c_slot)
        aj = jnp.sum(r_slot)
        fj = jnp.sum(cnf[pl.ds(j * L, L)])
        cnt_vec = jnp.where(sel, jnp.broadcast_to(cj, (L,)), cnt_vec)
        cor_vec = jnp.where(sel, jnp.broadcast_to(aj, (L,)), cor_vec)
        cnf_vec = jnp.where(sel, jnp.broadcast_to(fj, (L,)), cnf_vec)
        return (cnt_vec, cor_vec, cnf_vec)

    cnt_vec, cor_vec, cnf_vec = lax.fori_loop(
        0, N_BINS, fold_bin, (zeros, zeros, zeros)
    )
    row[pl.ds(0, L)] = cnt_vec
    row[pl.ds(L, L)] = cor_vec
    row[pl.ds(2 * L, L)] = cnf_vec

    pltpu.sync_copy(row, board.at[pl.ds(sidx * PROW, PROW)])
    plsc.subcore_barrier()

    @pl.when(sidx == 0)
    def _():
        pltpu.sync_copy(board, boardv)

        def add_tile(t, carry):
            c, a, f = carry
            o = t * PROW
            return (c + boardv[pl.ds(o, L)],
                    a + boardv[pl.ds(o + L, L)],
                    f + boardv[pl.ds(o + 2 * L, L)])

        c_sum, a_sum, f_sum = lax.fori_loop(
            0, NS, add_tile, (zeros, zeros, zeros)
        )
        row[pl.ds(0, L)] = c_sum
        row[pl.ds(L, L)] = a_sum
        row[pl.ds(2 * L, L)] = f_sum
        pltpu.sync_copy(row, rows_hbm.at[pl.ds(cidx * PROW, PROW)])
        pl.semaphore_signal(xsem, 1, device_id={"c": 1 - cidx, "s": 0})
        pl.semaphore_wait(xsem, 1)
        pltpu.sync_copy(rows_hbm.at[pl.ds((1 - cidx) * PROW, PROW)], other)

        counts_v = c_sum + other[pl.ds(0, L)]
        acc_v = a_sum + other[pl.ds(L, L)]
        conf_v = f_sum + other[pl.ds(2 * L, L)]
        ones = jnp.ones((L,), jnp.float32)
        ind = counts_v > jnp.float32(0.0)
        safe = jnp.where(ind, counts_v, ones)
        mean_acc = jnp.where(ind, acc_v / safe, acc_v)
        mean_conf = jnp.where(ind, conf_v / safe, conf_v)
        num = jnp.sum(counts_v * jnp.abs(mean_acc - mean_conf))
        tot = jnp.sum(counts_v)
        outv[...] = jnp.broadcast_to(num, (L,)) / jnp.broadcast_to(tot, (L,))

        @pl.when(cidx == 0)
        def _():
            pltpu.sync_copy(outv, ece_hbm)


@jax.jit
def kernel(yhs, phs, ys):
    bounds = jnp.linspace(0.0, 1.0, N_BINS + 1).astype(jnp.float32)
    _, ece_vec = _ece_hist_kernel(yhs, phs, ys, bounds)
    return ece_vec[0]
